# baseline (speedup 1.0000x reference)
import functools

import jax
import jax.numpy as jnp
from jax.experimental import pallas as pl
from jax.experimental.pallas import tpu as pltpu

D_MODEL = 1024
NUM_HEADS = 16
HEAD_DIM = 64
SEQ = 2048
TOPK = 128
INT_MIN = -2147483648

BQ = 512
NITER = 20
GRAN = 1 << (31 - NITER)


def _proj_kernel(x_ref, src_ref, wq_ref, wkv_ref, bq_ref, bkv_ref,
                 q_ref, kv_ref):
    x = x_ref[...]
    s = src_ref[...]
    q_ref[...] = jax.lax.dot_general(
        wq_ref[...], x, (((1,), (0,)), ((), ())),
        preferred_element_type=jnp.float32) + bq_ref[...]
    kv_ref[...] = jax.lax.dot_general(
        wkv_ref[...], s, (((1,), (0,)), ((), ())),
        preferred_element_type=jnp.float32) + bkv_ref[...]


def _select_probs(s):
    b = jax.lax.bitcast_convert_type(s, jnp.int32)
    key = jnp.where(b < 0, b ^ 0x7FFFFFFF, b)

    nchunk = SEQ // 128

    def count_true(cond16):
        ind = cond16.astype(jnp.int16)
        acc = ind[:, 0:128]
        for j in range(1, nchunk):
            acc = acc + ind[:, j * 128:(j + 1) * 128]
        return jnp.sum(acc.astype(jnp.int32), axis=1,
                       keepdims=True).astype(jnp.int16)

    t16 = (key >> 16).astype(jnp.int16)
    c16 = count_true(t16 >= 0)
    pos16 = c16 >= TOPK
    rank16 = TOPK - jnp.where(pos16, 0, c16)

    arr16 = jnp.where(pos16, t16, t16 ^ (-32768))

    def body_a(i, pfx):
        cand = pfx | jax.lax.convert_element_type(1 << (14 - i), jnp.int16)
        cnt = count_true(arr16 >= cand)
        return jnp.where(cnt >= rank16, cand, pfx)

    v16 = jax.lax.fori_loop(
        0, 15, body_a, jnp.zeros((s.shape[0], 1), jnp.int16))

    m_gt = count_true(arr16 > v16)
    rank2 = rank16 - m_gt
    match = arr16 == v16
    lo16 = key.astype(jnp.int16) ^ (-32768)
    arr_b = jnp.where(match, lo16, -32768)

    def body_b(i, pfxu):
        candu = pfxu | jax.lax.convert_element_type(1 << (15 - i), jnp.int16)
        dec = candu ^ (-32768)
        cnt = count_true(arr_b >= dec)
        return jnp.where(cnt >= rank2, candu, pfxu)

    pfxu = jax.lax.fori_loop(
        0, 8, body_b, jnp.zeros((s.shape[0], 1), jnp.int16))

    t16r = jnp.where(pos16, v16, v16 ^ (-32768))
    tkey = (t16r.astype(jnp.int32) << 16) | (pfxu.astype(jnp.int32) & 0xFFFF)

    mask = key >= tkey
    maskhi = key >= tkey + 256
    n_sel = jnp.sum(mask.astype(jnp.int32), axis=1, keepdims=True)
    m_hi = jnp.sum(maskhi.astype(jnp.int32), axis=1, keepdims=True)
    alpha = (TOPK - m_hi).astype(jnp.float32) / \
        jnp.maximum(n_sel - m_hi, 1).astype(jnp.float32)

    m = jnp.max(s, axis=1, keepdims=True)
    p = jnp.where(mask, jnp.exp(s - m), 0.0)
    p = jnp.where(maskhi, p, p * alpha)
    inv = 1.0 / jnp.sum(p, axis=1, keepdims=True)
    return p, inv


def _attn_kernel(q_ref, k_ref, v_ref, o_ref):
    qh = q_ref[0]
    kh = k_ref[0]
    vh = v_ref[0]
    hb = BQ // 2

    def scores(qpart):
        return jax.lax.dot_general(
            qpart, kh, (((0,), (0,)), ((), ())),
            preferred_element_type=jnp.float32) * (1.0 / 8.0)

    def pv(p):
        return jax.lax.dot_general(
            vh, p, (((1,), (1,)), ((), ())),
            preferred_element_type=jnp.float32)

    sa = scores(qh[:, :hb])
    sb = scores(qh[:, hb:])
    pa, inva = _select_probs(sa)
    oa = pv(pa)
    pb, invb = _select_probs(sb)
    ob = pv(pb)
    o_ref[0, :, 0:hb] = oa * inva[:, 0][None, :]
    o_ref[0, :, hb:BQ] = ob * invb[:, 0][None, :]


def _out_kernel(wm_ref, o_ref, bm_ref, y_ref):
    y_ref[...] = jax.lax.dot_general(
        wm_ref[...], o_ref[...], (((1,), (0,)), ((), ())),
        preferred_element_type=jnp.float32) + bm_ref[...]


def kernel(x, source, k, Wq, bq, Wk, bk, Wv, bv, Wm, bm):
    del k
    x2 = x[0]
    src2 = source[0]

    def rperm(W):
        return W.reshape(HEAD_DIM, NUM_HEADS, D_MODEL).transpose(1, 0, 2) \
                .reshape(D_MODEL, D_MODEL)

    def bperm(bvec):
        return bvec.reshape(HEAD_DIM, NUM_HEADS).T.reshape(D_MODEL, 1)

    Wq_p = rperm(Wq)
    Wkv_p = jnp.concatenate([rperm(Wk), rperm(Wv)], axis=0)
    bq_p = bperm(bq)
    bkv_p = jnp.concatenate([bperm(bk), bperm(bv)], axis=0)
    Wm_p = Wm.reshape(D_MODEL, HEAD_DIM, NUM_HEADS).transpose(0, 2, 1) \
             .reshape(D_MODEL, D_MODEL)

    q, kv = pl.pallas_call(
        _proj_kernel,
        out_shape=(
            jax.ShapeDtypeStruct((D_MODEL, SEQ), jnp.float32),
            jax.ShapeDtypeStruct((2 * D_MODEL, SEQ), jnp.float32),
        ),
    )(x2, src2, Wq_p, Wkv_p, bq_p, bkv_p)

    q3 = q.reshape(NUM_HEADS, HEAD_DIM, SEQ)
    k3 = kv[:D_MODEL].reshape(NUM_HEADS, HEAD_DIM, SEQ)
    v3 = kv[D_MODEL:].reshape(NUM_HEADS, HEAD_DIM, SEQ)

    nq = SEQ // BQ
    o3 = pl.pallas_call(
        _attn_kernel,
        grid=(NUM_HEADS, nq),
        in_specs=[
            pl.BlockSpec((1, HEAD_DIM, BQ), lambda h, qi: (h, 0, qi)),
            pl.BlockSpec((1, HEAD_DIM, SEQ), lambda h, qi: (h, 0, 0)),
            pl.BlockSpec((1, HEAD_DIM, SEQ), lambda h, qi: (h, 0, 0)),
        ],
        out_specs=pl.BlockSpec((1, HEAD_DIM, BQ), lambda h, qi: (h, 0, qi)),
        out_shape=jax.ShapeDtypeStruct((NUM_HEADS, HEAD_DIM, SEQ),
                                       jnp.float32),
    )(q3, k3, v3)

    o2 = o3.reshape(D_MODEL, SEQ)
    y = pl.pallas_call(
        _out_kernel,
        out_shape=jax.ShapeDtypeStruct((D_MODEL, SEQ), jnp.float32),
    )(Wm_p, o2, bm.reshape(D_MODEL, 1))
    return y[None]


# speedup vs baseline: 1.2250x; 1.2250x over previous
import functools

import jax
import jax.numpy as jnp
from jax.experimental import pallas as pl
from jax.experimental.pallas import tpu as pltpu

D_MODEL = 1024
NUM_HEADS = 16
HEAD_DIM = 64
SEQ = 2048
TOPK = 128
INT_MIN = -2147483648

BQ = 1024
NITER = 20
GRAN = 1 << (31 - NITER)


def _proj_kernel(x_ref, src_ref, wq_ref, wkv_ref, bq_ref, bkv_ref,
                 q_ref, kv_ref):
    x = x_ref[...]
    s = src_ref[...]
    q_ref[...] = jax.lax.dot_general(
        wq_ref[...], x, (((1,), (0,)), ((), ())),
        preferred_element_type=jnp.float32) + bq_ref[...]
    kv_ref[...] = jax.lax.dot_general(
        wkv_ref[...], s, (((1,), (0,)), ((), ())),
        preferred_element_type=jnp.float32) + bkv_ref[...]


def _select_probs(s):
    b = jax.lax.bitcast_convert_type(s, jnp.int32)
    key = jnp.where(b < 0, b ^ 0x7FFFFFFF, b)

    nchunk = SEQ // 128

    def count_true(cond16):
        ind = cond16.astype(jnp.int16)
        acc = ind[:, 0:128]
        for j in range(1, nchunk):
            acc = acc + ind[:, j * 128:(j + 1) * 128]
        return jnp.sum(acc.astype(jnp.int32), axis=1,
                       keepdims=True).astype(jnp.int16)

    t16 = (key >> 16).astype(jnp.int16)
    c16 = count_true(t16 >= 0)
    pos16 = c16 >= TOPK
    rank16 = TOPK - jnp.where(pos16, 0, c16)

    arr16 = jnp.where(pos16, t16, t16 ^ (-32768))

    def body_a(i, pfx):
        cand = pfx | jax.lax.convert_element_type(1 << (14 - i), jnp.int16)
        cnt = count_true(arr16 >= cand)
        return jnp.where(cnt >= rank16, cand, pfx)

    v16 = jax.lax.fori_loop(
        0, 15, body_a, jnp.zeros((s.shape[0], 1), jnp.int16))

    m_gt = count_true(arr16 > v16)
    rank2 = rank16 - m_gt
    match = arr16 == v16
    lo16 = key.astype(jnp.int16) ^ (-32768)
    arr_b = jnp.where(match, lo16, -32768)

    def body_b(i, pfxu):
        candu = pfxu | jax.lax.convert_element_type(1 << (15 - i), jnp.int16)
        dec = candu ^ (-32768)
        cnt = count_true(arr_b >= dec)
        return jnp.where(cnt >= rank2, candu, pfxu)

    pfxu = jax.lax.fori_loop(
        0, 8, body_b, jnp.zeros((s.shape[0], 1), jnp.int16))

    t16r = jnp.where(pos16, v16, v16 ^ (-32768))
    tkey = (t16r.astype(jnp.int32) << 16) | (pfxu.astype(jnp.int32) & 0xFFFF)

    mask = key >= tkey
    maskhi = key >= tkey + 256
    n_sel = jnp.sum(mask.astype(jnp.int32), axis=1, keepdims=True)
    m_hi = jnp.sum(maskhi.astype(jnp.int32), axis=1, keepdims=True)
    alpha = (TOPK - m_hi).astype(jnp.float32) / \
        jnp.maximum(n_sel - m_hi, 1).astype(jnp.float32)

    m = jnp.max(s, axis=1, keepdims=True)
    p = jnp.where(mask, jnp.exp(s - m), 0.0)
    p = jnp.where(maskhi, p, p * alpha)
    inv = 1.0 / jnp.sum(p, axis=1, keepdims=True)
    return p, inv


def _attn_kernel(q_ref, k_ref, v_ref, o_ref):
    qh = q_ref[0]
    kh = k_ref[0]
    s = jax.lax.dot_general(
        qh, kh, (((0,), (0,)), ((), ())),
        preferred_element_type=jnp.float32) * (1.0 / 8.0)
    p, inv = _select_probs(s)
    o = jax.lax.dot_general(
        v_ref[0], p, (((1,), (1,)), ((), ())),
        preferred_element_type=jnp.float32)
    o_ref[0] = o * inv[:, 0][None, :]


def _out_kernel(wm_ref, o_ref, bm_ref, y_ref):
    y_ref[...] = jax.lax.dot_general(
        wm_ref[...], o_ref[...], (((1,), (0,)), ((), ())),
        preferred_element_type=jnp.float32) + bm_ref[...]


def kernel(x, source, k, Wq, bq, Wk, bk, Wv, bv, Wm, bm):
    del k
    x2 = x[0]
    src2 = source[0]

    def rperm(W):
        return W.reshape(HEAD_DIM, NUM_HEADS, D_MODEL).transpose(1, 0, 2) \
                .reshape(D_MODEL, D_MODEL)

    def bperm(bvec):
        return bvec.reshape(HEAD_DIM, NUM_HEADS).T.reshape(D_MODEL, 1)

    Wq_p = rperm(Wq)
    Wkv_p = jnp.concatenate([rperm(Wk), rperm(Wv)], axis=0)
    bq_p = bperm(bq)
    bkv_p = jnp.concatenate([bperm(bk), bperm(bv)], axis=0)
    Wm_p = Wm.reshape(D_MODEL, HEAD_DIM, NUM_HEADS).transpose(0, 2, 1) \
             .reshape(D_MODEL, D_MODEL)

    q, kv = pl.pallas_call(
        _proj_kernel,
        out_shape=(
            jax.ShapeDtypeStruct((D_MODEL, SEQ), jnp.float32),
            jax.ShapeDtypeStruct((2 * D_MODEL, SEQ), jnp.float32),
        ),
    )(x2, src2, Wq_p, Wkv_p, bq_p, bkv_p)

    q3 = q.reshape(NUM_HEADS, HEAD_DIM, SEQ)
    k3 = kv[:D_MODEL].reshape(NUM_HEADS, HEAD_DIM, SEQ)
    v3 = kv[D_MODEL:].reshape(NUM_HEADS, HEAD_DIM, SEQ)

    nq = SEQ // BQ
    o3 = pl.pallas_call(
        _attn_kernel,
        grid=(NUM_HEADS, nq),
        in_specs=[
            pl.BlockSpec((1, HEAD_DIM, BQ), lambda h, qi: (h, 0, qi)),
            pl.BlockSpec((1, HEAD_DIM, SEQ), lambda h, qi: (h, 0, 0)),
            pl.BlockSpec((1, HEAD_DIM, SEQ), lambda h, qi: (h, 0, 0)),
        ],
        out_specs=pl.BlockSpec((1, HEAD_DIM, BQ), lambda h, qi: (h, 0, qi)),
        out_shape=jax.ShapeDtypeStruct((NUM_HEADS, HEAD_DIM, SEQ),
                                       jnp.float32),
    )(q3, k3, v3)

    o2 = o3.reshape(D_MODEL, SEQ)
    y = pl.pallas_call(
        _out_kernel,
        out_shape=jax.ShapeDtypeStruct((D_MODEL, SEQ), jnp.float32),
    )(Wm_p, o2, bm.reshape(D_MODEL, 1))
    return y[None]


# speedup vs baseline: 1.2544x; 1.0240x over previous
import functools

import jax
import jax.numpy as jnp
from jax.experimental import pallas as pl
from jax.experimental.pallas import tpu as pltpu

D_MODEL = 1024
NUM_HEADS = 16
HEAD_DIM = 64
SEQ = 2048
TOPK = 128
INT_MIN = -2147483648

BQ = 2048
NITER = 20
GRAN = 1 << (31 - NITER)


def _proj_kernel(x_ref, src_ref, wq_ref, wkv_ref, bq_ref, bkv_ref,
                 q_ref, kv_ref):
    x = x_ref[...]
    s = src_ref[...]
    q_ref[...] = jax.lax.dot_general(
        wq_ref[...], x, (((1,), (0,)), ((), ())),
        preferred_element_type=jnp.float32) + bq_ref[...]
    kv_ref[...] = jax.lax.dot_general(
        wkv_ref[...], s, (((1,), (0,)), ((), ())),
        preferred_element_type=jnp.float32) + bkv_ref[...]


def _select_probs(s):
    b = jax.lax.bitcast_convert_type(s, jnp.int32)
    key = jnp.where(b < 0, b ^ 0x7FFFFFFF, b)

    nchunk = SEQ // 128

    def count_true(cond16):
        ind = cond16.astype(jnp.int16)
        acc = ind[:, 0:128]
        for j in range(1, nchunk):
            acc = acc + ind[:, j * 128:(j + 1) * 128]
        return jnp.sum(acc.astype(jnp.int32), axis=1,
                       keepdims=True).astype(jnp.int16)

    t16 = (key >> 16).astype(jnp.int16)
    c16 = count_true(t16 >= 0)
    pos16 = c16 >= TOPK
    rank16 = TOPK - jnp.where(pos16, 0, c16)

    arr16 = jnp.where(pos16, t16, t16 ^ (-32768))

    def body_a(i, pfx):
        cand = pfx | jax.lax.convert_element_type(1 << (14 - i), jnp.int16)
        cnt = count_true(arr16 >= cand)
        return jnp.where(cnt >= rank16, cand, pfx)

    v16 = jax.lax.fori_loop(
        0, 15, body_a, jnp.zeros((s.shape[0], 1), jnp.int16))

    m_gt = count_true(arr16 > v16)
    rank2 = rank16 - m_gt
    match = arr16 == v16
    lo16 = key.astype(jnp.int16) ^ (-32768)
    arr_b = jnp.where(match, lo16, -32768)

    def body_b(i, pfxu):
        candu = pfxu | jax.lax.convert_element_type(1 << (15 - i), jnp.int16)
        dec = candu ^ (-32768)
        cnt = count_true(arr_b >= dec)
        return jnp.where(cnt >= rank2, candu, pfxu)

    pfxu = jax.lax.fori_loop(
        0, 8, body_b, jnp.zeros((s.shape[0], 1), jnp.int16))

    t16r = jnp.where(pos16, v16, v16 ^ (-32768))
    tkey = (t16r.astype(jnp.int32) << 16) | (pfxu.astype(jnp.int32) & 0xFFFF)

    mask = key >= tkey
    maskhi = key >= tkey + 256
    n_sel = jnp.sum(mask.astype(jnp.int32), axis=1, keepdims=True)
    m_hi = jnp.sum(maskhi.astype(jnp.int32), axis=1, keepdims=True)
    alpha = (TOPK - m_hi).astype(jnp.float32) / \
        jnp.maximum(n_sel - m_hi, 1).astype(jnp.float32)

    m = jnp.max(s, axis=1, keepdims=True)
    p = jnp.where(mask, jnp.exp(s - m), 0.0)
    p = jnp.where(maskhi, p, p * alpha)
    inv = 1.0 / jnp.sum(p, axis=1, keepdims=True)
    return p, inv


def _attn_kernel(q_ref, k_ref, v_ref, o_ref):
    qh = q_ref[0]
    kh = k_ref[0]
    s = jax.lax.dot_general(
        qh, kh, (((0,), (0,)), ((), ())),
        preferred_element_type=jnp.float32)
    p, inv = _select_probs(s)
    o = jax.lax.dot_general(
        v_ref[0], p, (((1,), (1,)), ((), ())),
        preferred_element_type=jnp.float32)
    o_ref[0] = o * inv[:, 0][None, :]


def _out_kernel(wm_ref, o_ref, bm_ref, y_ref):
    y_ref[...] = jax.lax.dot_general(
        wm_ref[...], o_ref[...], (((1,), (0,)), ((), ())),
        preferred_element_type=jnp.float32) + bm_ref[...]


def kernel(x, source, k, Wq, bq, Wk, bk, Wv, bv, Wm, bm):
    del k
    x2 = x[0]
    src2 = source[0]

    def rperm(W):
        return W.reshape(HEAD_DIM, NUM_HEADS, D_MODEL).transpose(1, 0, 2) \
                .reshape(D_MODEL, D_MODEL)

    def bperm(bvec):
        return bvec.reshape(HEAD_DIM, NUM_HEADS).T.reshape(D_MODEL, 1)

    Wq_p = rperm(Wq) * 0.125
    Wkv_p = jnp.concatenate([rperm(Wk), rperm(Wv)], axis=0)
    bq_p = bperm(bq) * 0.125
    bkv_p = jnp.concatenate([bperm(bk), bperm(bv)], axis=0)
    Wm_p = Wm.reshape(D_MODEL, HEAD_DIM, NUM_HEADS).transpose(0, 2, 1) \
             .reshape(D_MODEL, D_MODEL)

    q, kv = pl.pallas_call(
        _proj_kernel,
        out_shape=(
            jax.ShapeDtypeStruct((D_MODEL, SEQ), jnp.float32),
            jax.ShapeDtypeStruct((2 * D_MODEL, SEQ), jnp.float32),
        ),
    )(x2, src2, Wq_p, Wkv_p, bq_p, bkv_p)

    q3 = q.reshape(NUM_HEADS, HEAD_DIM, SEQ)
    k3 = kv[:D_MODEL].reshape(NUM_HEADS, HEAD_DIM, SEQ)
    v3 = kv[D_MODEL:].reshape(NUM_HEADS, HEAD_DIM, SEQ)

    nq = SEQ // BQ
    o3 = pl.pallas_call(
        _attn_kernel,
        grid=(NUM_HEADS, nq),
        in_specs=[
            pl.BlockSpec((1, HEAD_DIM, BQ), lambda h, qi: (h, 0, qi)),
            pl.BlockSpec((1, HEAD_DIM, SEQ), lambda h, qi: (h, 0, 0)),
            pl.BlockSpec((1, HEAD_DIM, SEQ), lambda h, qi: (h, 0, 0)),
        ],
        out_specs=pl.BlockSpec((1, HEAD_DIM, BQ), lambda h, qi: (h, 0, qi)),
        out_shape=jax.ShapeDtypeStruct((NUM_HEADS, HEAD_DIM, SEQ),
                                       jnp.float32),
    )(q3, k3, v3)

    o2 = o3.reshape(D_MODEL, SEQ)
    y = pl.pallas_call(
        _out_kernel,
        out_shape=jax.ShapeDtypeStruct((D_MODEL, SEQ), jnp.float32),
    )(Wm_p, o2, bm.reshape(D_MODEL, 1))
    return y[None]


# speedup vs baseline: 1.3641x; 1.0875x over previous
import functools

import jax
import jax.numpy as jnp
from jax.experimental import pallas as pl
from jax.experimental.pallas import tpu as pltpu

D_MODEL = 1024
NUM_HEADS = 16
HEAD_DIM = 64
SEQ = 2048
TOPK = 128
INT_MIN = -2147483648

BQ = 2048
NITER = 20
GRAN = 1 << (31 - NITER)


def _proj_kernel(x_ref, src_ref, wq_ref, wkv_ref, bq_ref, bkv_ref,
                 q_ref, kv_ref):
    x = x_ref[...]
    s = src_ref[...]
    q_ref[...] = jax.lax.dot_general(
        wq_ref[...], x, (((1,), (0,)), ((), ())),
        preferred_element_type=jnp.float32) + bq_ref[...]
    kv_ref[...] = jax.lax.dot_general(
        wkv_ref[...], s, (((1,), (0,)), ((), ())),
        preferred_element_type=jnp.float32) + bkv_ref[...]


def _select_probs(s):
    b = jax.lax.bitcast_convert_type(s, jnp.int32)
    key = jnp.where(b < 0, b ^ 0x7FFFFFFF, b)

    nchunk = SEQ // 128

    ones128 = jnp.ones((128, 128), jnp.bfloat16)

    def count_true(cond16):
        ind = cond16.astype(jnp.int16)
        acc = ind[:, 0:128]
        for j in range(1, nchunk):
            acc = acc + ind[:, j * 128:(j + 1) * 128]
        cnt = jax.lax.dot_general(
            acc.astype(jnp.bfloat16), ones128, (((1,), (0,)), ((), ())),
            preferred_element_type=jnp.float32)[:, :1]
        return cnt.astype(jnp.int16)

    t16 = (key >> 16).astype(jnp.int16)
    c16 = count_true(t16 >= 0)
    pos16 = c16 >= TOPK
    rank16 = TOPK - jnp.where(pos16, 0, c16)

    arr16 = jnp.where(pos16, t16, t16 ^ (-32768))

    def body_a(i, pfx):
        cand = pfx | jax.lax.convert_element_type(1 << (14 - i), jnp.int16)
        cnt = count_true(arr16 >= cand)
        return jnp.where(cnt >= rank16, cand, pfx)

    v16 = jax.lax.fori_loop(
        0, 15, body_a, jnp.zeros((s.shape[0], 1), jnp.int16))

    m_gt = count_true(arr16 > v16)
    rank2 = rank16 - m_gt
    match = arr16 == v16
    lo16 = key.astype(jnp.int16) ^ (-32768)
    arr_b = jnp.where(match, lo16, -32768)

    def body_b(i, pfxu):
        candu = pfxu | jax.lax.convert_element_type(1 << (15 - i), jnp.int16)
        dec = candu ^ (-32768)
        cnt = count_true(arr_b >= dec)
        return jnp.where(cnt >= rank2, candu, pfxu)

    pfxu = jax.lax.fori_loop(
        0, 8, body_b, jnp.zeros((s.shape[0], 1), jnp.int16))

    t16r = jnp.where(pos16, v16, v16 ^ (-32768))
    tkey = (t16r.astype(jnp.int32) << 16) | (pfxu.astype(jnp.int32) & 0xFFFF)

    mask = key >= tkey
    maskhi = key >= tkey + 256
    n_sel = jnp.sum(mask.astype(jnp.int32), axis=1, keepdims=True)
    m_hi = jnp.sum(maskhi.astype(jnp.int32), axis=1, keepdims=True)
    alpha = (TOPK - m_hi).astype(jnp.float32) / \
        jnp.maximum(n_sel - m_hi, 1).astype(jnp.float32)

    m = jnp.max(s, axis=1, keepdims=True)
    p = jnp.where(mask, jnp.exp(s - m), 0.0)
    p = jnp.where(maskhi, p, p * alpha)
    inv = 1.0 / jnp.sum(p, axis=1, keepdims=True)
    return p, inv


def _attn_kernel(q_ref, k_ref, v_ref, o_ref):
    qh = q_ref[0]
    kh = k_ref[0]
    s = jax.lax.dot_general(
        qh, kh, (((0,), (0,)), ((), ())),
        preferred_element_type=jnp.float32)
    p, inv = _select_probs(s)
    o = jax.lax.dot_general(
        v_ref[0], p, (((1,), (1,)), ((), ())),
        preferred_element_type=jnp.float32)
    o_ref[0] = o * inv[:, 0][None, :]


def _out_kernel(wm_ref, o_ref, bm_ref, y_ref):
    y_ref[...] = jax.lax.dot_general(
        wm_ref[...], o_ref[...], (((1,), (0,)), ((), ())),
        preferred_element_type=jnp.float32) + bm_ref[...]


def kernel(x, source, k, Wq, bq, Wk, bk, Wv, bv, Wm, bm):
    del k
    x2 = x[0]
    src2 = source[0]

    def rperm(W):
        return W.reshape(HEAD_DIM, NUM_HEADS, D_MODEL).transpose(1, 0, 2) \
                .reshape(D_MODEL, D_MODEL)

    def bperm(bvec):
        return bvec.reshape(HEAD_DIM, NUM_HEADS).T.reshape(D_MODEL, 1)

    Wq_p = rperm(Wq) * 0.125
    Wkv_p = jnp.concatenate([rperm(Wk), rperm(Wv)], axis=0)
    bq_p = bperm(bq) * 0.125
    bkv_p = jnp.concatenate([bperm(bk), bperm(bv)], axis=0)
    Wm_p = Wm.reshape(D_MODEL, HEAD_DIM, NUM_HEADS).transpose(0, 2, 1) \
             .reshape(D_MODEL, D_MODEL)

    q, kv = pl.pallas_call(
        _proj_kernel,
        out_shape=(
            jax.ShapeDtypeStruct((D_MODEL, SEQ), jnp.float32),
            jax.ShapeDtypeStruct((2 * D_MODEL, SEQ), jnp.float32),
        ),
    )(x2, src2, Wq_p, Wkv_p, bq_p, bkv_p)

    q3 = q.reshape(NUM_HEADS, HEAD_DIM, SEQ)
    k3 = kv[:D_MODEL].reshape(NUM_HEADS, HEAD_DIM, SEQ)
    v3 = kv[D_MODEL:].reshape(NUM_HEADS, HEAD_DIM, SEQ)

    nq = SEQ // BQ
    o3 = pl.pallas_call(
        _attn_kernel,
        grid=(NUM_HEADS, nq),
        in_specs=[
            pl.BlockSpec((1, HEAD_DIM, BQ), lambda h, qi: (h, 0, qi)),
            pl.BlockSpec((1, HEAD_DIM, SEQ), lambda h, qi: (h, 0, 0)),
            pl.BlockSpec((1, HEAD_DIM, SEQ), lambda h, qi: (h, 0, 0)),
        ],
        out_specs=pl.BlockSpec((1, HEAD_DIM, BQ), lambda h, qi: (h, 0, qi)),
        out_shape=jax.ShapeDtypeStruct((NUM_HEADS, HEAD_DIM, SEQ),
                                       jnp.float32),
    )(q3, k3, v3)

    o2 = o3.reshape(D_MODEL, SEQ)
    y = pl.pallas_call(
        _out_kernel,
        out_shape=jax.ShapeDtypeStruct((D_MODEL, SEQ), jnp.float32),
    )(Wm_p, o2, bm.reshape(D_MODEL, 1))
    return y[None]


# speedup vs baseline: 1.3732x; 1.0067x over previous
import functools

import jax
import jax.numpy as jnp
from jax.experimental import pallas as pl
from jax.experimental.pallas import tpu as pltpu

D_MODEL = 1024
NUM_HEADS = 16
HEAD_DIM = 64
SEQ = 2048
TOPK = 128
INT_MIN = -2147483648

BQ = 2048
NITER = 20
GRAN = 1 << (31 - NITER)


def _proj_kernel(x_ref, src_ref, wq_ref, wkv_ref, bq_ref, bkv_ref,
                 q_ref, kv_ref):
    x = x_ref[...]
    s = src_ref[...]
    q_ref[...] = jax.lax.dot_general(
        wq_ref[...], x, (((1,), (0,)), ((), ())),
        preferred_element_type=jnp.float32) + bq_ref[...]
    kv_ref[...] = jax.lax.dot_general(
        wkv_ref[...], s, (((1,), (0,)), ((), ())),
        preferred_element_type=jnp.float32) + bkv_ref[...]


def _select_probs(s):
    b = jax.lax.bitcast_convert_type(s, jnp.int32)
    key = jnp.where(b < 0, b ^ 0x7FFFFFFF, b)

    nchunk = SEQ // 128

    ones128 = jnp.ones((128, 128), jnp.bfloat16)

    def count_true(cond16):
        ind = cond16.astype(jnp.int16)
        acc = ind[:, 0:128]
        for j in range(1, nchunk):
            acc = acc + ind[:, j * 128:(j + 1) * 128]
        cnt = jax.lax.dot_general(
            acc.astype(jnp.bfloat16), ones128, (((1,), (0,)), ((), ())),
            preferred_element_type=jnp.float32)[:, :1]
        return cnt.astype(jnp.int16)

    t16 = (key >> 16).astype(jnp.int16)
    c16 = count_true(t16 >= 0)
    pos16 = c16 >= TOPK
    rank16 = TOPK - jnp.where(pos16, 0, c16)

    arr16 = jnp.where(pos16, t16, t16 ^ (-32768))

    def body_a(i, pfx):
        cand = pfx | jax.lax.convert_element_type(1 << (14 - i), jnp.int16)
        cnt = count_true(arr16 >= cand)
        return jnp.where(cnt >= rank16, cand, pfx)

    v16 = jax.lax.fori_loop(
        0, 15, body_a, jnp.zeros((s.shape[0], 1), jnp.int16))

    m_gt = count_true(arr16 > v16)
    rank2 = rank16 - m_gt
    match = arr16 == v16
    lo16 = key.astype(jnp.int16) ^ (-32768)
    arr_b = jnp.where(match, lo16, -32768)

    def body_b(i, pfxu):
        candu = pfxu | jax.lax.convert_element_type(1 << (15 - i), jnp.int16)
        dec = candu ^ (-32768)
        cnt = count_true(arr_b >= dec)
        return jnp.where(cnt >= rank2, candu, pfxu)

    pfxu = jax.lax.fori_loop(
        0, 8, body_b, jnp.zeros((s.shape[0], 1), jnp.int16))

    t16r = jnp.where(pos16, v16, v16 ^ (-32768))
    tkey = (t16r.astype(jnp.int32) << 16) | (pfxu.astype(jnp.int32) & 0xFFFF)

    mask = key >= tkey
    maskhi = key >= tkey + 256
    n_sel = count_true(mask)
    m_hi = count_true(maskhi)
    alpha = (TOPK - m_hi).astype(jnp.float32) / \
        jnp.maximum((n_sel - m_hi).astype(jnp.float32), 1.0)

    m = jnp.max(s, axis=1, keepdims=True)
    p = jnp.where(mask, jnp.exp(s - m), 0.0)
    p = jnp.where(maskhi, p, p * alpha)
    inv = 1.0 / jnp.sum(p, axis=1, keepdims=True)
    return p, inv


def _attn_kernel(q_ref, k_ref, v_ref, o_ref):
    qh = q_ref[0]
    kh = k_ref[0]
    s = jax.lax.dot_general(
        qh, kh, (((0,), (0,)), ((), ())),
        preferred_element_type=jnp.float32)
    p, inv = _select_probs(s)
    o = jax.lax.dot_general(
        v_ref[0], p, (((1,), (1,)), ((), ())),
        preferred_element_type=jnp.float32)
    o_ref[0] = o * inv[:, 0][None, :]


def _out_kernel(wm_ref, o_ref, bm_ref, y_ref):
    y_ref[...] = jax.lax.dot_general(
        wm_ref[...], o_ref[...], (((1,), (0,)), ((), ())),
        preferred_element_type=jnp.float32) + bm_ref[...]


def kernel(x, source, k, Wq, bq, Wk, bk, Wv, bv, Wm, bm):
    del k
    x2 = x[0]
    src2 = source[0]

    def rperm(W):
        return W.reshape(HEAD_DIM, NUM_HEADS, D_MODEL).transpose(1, 0, 2) \
                .reshape(D_MODEL, D_MODEL)

    def bperm(bvec):
        return bvec.reshape(HEAD_DIM, NUM_HEADS).T.reshape(D_MODEL, 1)

    Wq_p = rperm(Wq) * 0.125
    Wkv_p = jnp.concatenate([rperm(Wk), rperm(Wv)], axis=0)
    bq_p = bperm(bq) * 0.125
    bkv_p = jnp.concatenate([bperm(bk), bperm(bv)], axis=0)
    Wm_p = Wm.reshape(D_MODEL, HEAD_DIM, NUM_HEADS).transpose(0, 2, 1) \
             .reshape(D_MODEL, D_MODEL)

    q, kv = pl.pallas_call(
        _proj_kernel,
        out_shape=(
            jax.ShapeDtypeStruct((D_MODEL, SEQ), jnp.float32),
            jax.ShapeDtypeStruct((2 * D_MODEL, SEQ), jnp.float32),
        ),
    )(x2, src2, Wq_p, Wkv_p, bq_p, bkv_p)

    q3 = q.reshape(NUM_HEADS, HEAD_DIM, SEQ)
    k3 = kv[:D_MODEL].reshape(NUM_HEADS, HEAD_DIM, SEQ)
    v3 = kv[D_MODEL:].reshape(NUM_HEADS, HEAD_DIM, SEQ)

    nq = SEQ // BQ
    o3 = pl.pallas_call(
        _attn_kernel,
        grid=(NUM_HEADS, nq),
        in_specs=[
            pl.BlockSpec((1, HEAD_DIM, BQ), lambda h, qi: (h, 0, qi)),
            pl.BlockSpec((1, HEAD_DIM, SEQ), lambda h, qi: (h, 0, 0)),
            pl.BlockSpec((1, HEAD_DIM, SEQ), lambda h, qi: (h, 0, 0)),
        ],
        out_specs=pl.BlockSpec((1, HEAD_DIM, BQ), lambda h, qi: (h, 0, qi)),
        out_shape=jax.ShapeDtypeStruct((NUM_HEADS, HEAD_DIM, SEQ),
                                       jnp.float32),
    )(q3, k3, v3)

    o2 = o3.reshape(D_MODEL, SEQ)
    y = pl.pallas_call(
        _out_kernel,
        out_shape=jax.ShapeDtypeStruct((D_MODEL, SEQ), jnp.float32),
    )(Wm_p, o2, bm.reshape(D_MODEL, 1))
    return y[None]


# speedup vs baseline: 1.3734x; 1.0001x over previous
import jax
import jax.numpy as jnp
from jax.experimental import pallas as pl
from jax.experimental.pallas import tpu as pltpu

D_MODEL = 1024
NUM_HEADS = 16
HEAD_DIM = 64
SEQ = 2048
TOPK = 128

BQ = 2048


def _proj_kernel(x_ref, src_ref, wq_ref, wkv_ref, bq_ref, bkv_ref,
                 q_ref, kv_ref):
    x = x_ref[...]
    s = src_ref[...]
    q_ref[...] = jax.lax.dot_general(
        wq_ref[...], x, (((1,), (0,)), ((), ())),
        preferred_element_type=jnp.float32) + bq_ref[...]
    kv_ref[...] = jax.lax.dot_general(
        wkv_ref[...], s, (((1,), (0,)), ((), ())),
        preferred_element_type=jnp.float32) + bkv_ref[...]


def _select_probs(s):
    b = jax.lax.bitcast_convert_type(s, jnp.int32)
    key = jnp.where(b < 0, b ^ 0x7FFFFFFF, b)

    nchunk = SEQ // 128

    ones128 = jnp.ones((128, 128), jnp.bfloat16)

    def count_true(cond16):
        ind = cond16.astype(jnp.int16)
        acc = ind[:, 0:128]
        for j in range(1, nchunk):
            acc = acc + ind[:, j * 128:(j + 1) * 128]
        cnt = jax.lax.dot_general(
            acc.astype(jnp.bfloat16), ones128, (((1,), (0,)), ((), ())),
            preferred_element_type=jnp.float32)[:, :1]
        return cnt.astype(jnp.int16)

    t16 = (key >> 16).astype(jnp.int16)
    c16 = count_true(t16 >= 0)
    pos16 = c16 >= TOPK
    rank16 = TOPK - jnp.where(pos16, 0, c16)

    arr16 = jnp.where(pos16, t16, t16 ^ (-32768))

    def body_a(i, pfx):
        cand = pfx | jax.lax.convert_element_type(1 << (14 - i), jnp.int16)
        cnt = count_true(arr16 >= cand)
        return jnp.where(cnt >= rank16, cand, pfx)

    v16 = jax.lax.fori_loop(
        0, 15, body_a, jnp.zeros((s.shape[0], 1), jnp.int16))

    m_gt = count_true(arr16 > v16)
    rank2 = rank16 - m_gt
    match = arr16 == v16
    lo16 = key.astype(jnp.int16) ^ (-32768)
    arr_b = jnp.where(match, lo16, -32768)

    def body_b(i, pfxu):
        candu = pfxu | jax.lax.convert_element_type(1 << (15 - i), jnp.int16)
        dec = candu ^ (-32768)
        cnt = count_true(arr_b >= dec)
        return jnp.where(cnt >= rank2, candu, pfxu)

    pfxu = jax.lax.fori_loop(
        0, 8, body_b, jnp.zeros((s.shape[0], 1), jnp.int16))

    t16r = jnp.where(pos16, v16, v16 ^ (-32768))
    tkey = (t16r.astype(jnp.int32) << 16) | (pfxu.astype(jnp.int32) & 0xFFFF)

    mask = key >= tkey
    maskhi = key >= tkey + 256
    n_sel = count_true(mask)
    m_hi = count_true(maskhi)
    alpha = (TOPK - m_hi).astype(jnp.float32) / \
        jnp.maximum((n_sel - m_hi).astype(jnp.float32), 1.0)

    m = jnp.max(s, axis=1, keepdims=True)
    p = jnp.where(mask, jnp.exp(s - m), 0.0)
    p = jnp.where(maskhi, p, p * alpha)
    inv = 1.0 / jnp.sum(p, axis=1, keepdims=True)
    return p, inv


def _attn_kernel(q_ref, k_ref, v_ref, o_ref):
    qh = q_ref[0]
    kh = k_ref[0]
    s = jax.lax.dot_general(
        qh, kh, (((0,), (0,)), ((), ())),
        preferred_element_type=jnp.float32)
    p, inv = _select_probs(s)
    o = jax.lax.dot_general(
        v_ref[0], p, (((1,), (1,)), ((), ())),
        preferred_element_type=jnp.float32)
    o_ref[0] = o * inv[:, 0][None, :]


def _out_kernel(wm_ref, o_ref, bm_ref, y_ref):
    y_ref[...] = jax.lax.dot_general(
        wm_ref[...], o_ref[...], (((1,), (0,)), ((), ())),
        preferred_element_type=jnp.float32) + bm_ref[...]


def kernel(x, source, k, Wq, bq, Wk, bk, Wv, bv, Wm, bm):
    del k
    x2 = x[0]
    src2 = source[0]

    def rperm(W):
        return W.reshape(HEAD_DIM, NUM_HEADS, D_MODEL).transpose(1, 0, 2) \
                .reshape(D_MODEL, D_MODEL)

    def bperm(bvec):
        return bvec.reshape(HEAD_DIM, NUM_HEADS).T.reshape(D_MODEL, 1)

    Wq_p = rperm(Wq) * 0.125
    Wkv_p = jnp.concatenate([rperm(Wk), rperm(Wv)], axis=0)
    bq_p = bperm(bq) * 0.125
    bkv_p = jnp.concatenate([bperm(bk), bperm(bv)], axis=0)
    Wm_p = Wm.reshape(D_MODEL, HEAD_DIM, NUM_HEADS).transpose(0, 2, 1) \
             .reshape(D_MODEL, D_MODEL)

    q, kv = pl.pallas_call(
        _proj_kernel,
        out_shape=(
            jax.ShapeDtypeStruct((D_MODEL, SEQ), jnp.float32),
            jax.ShapeDtypeStruct((2 * D_MODEL, SEQ), jnp.float32),
        ),
    )(x2, src2, Wq_p, Wkv_p, bq_p, bkv_p)

    q3 = q.reshape(NUM_HEADS, HEAD_DIM, SEQ)
    k3 = kv[:D_MODEL].reshape(NUM_HEADS, HEAD_DIM, SEQ)
    v3 = kv[D_MODEL:].reshape(NUM_HEADS, HEAD_DIM, SEQ)

    nq = SEQ // BQ
    o3 = pl.pallas_call(
        _attn_kernel,
        grid=(NUM_HEADS, nq),
        in_specs=[
            pl.BlockSpec((1, HEAD_DIM, BQ), lambda h, qi: (h, 0, qi)),
            pl.BlockSpec((1, HEAD_DIM, SEQ), lambda h, qi: (h, 0, 0)),
            pl.BlockSpec((1, HEAD_DIM, SEQ), lambda h, qi: (h, 0, 0)),
        ],
        out_specs=pl.BlockSpec((1, HEAD_DIM, BQ), lambda h, qi: (h, 0, qi)),
        out_shape=jax.ShapeDtypeStruct((NUM_HEADS, HEAD_DIM, SEQ),
                                       jnp.float32),
    )(q3, k3, v3)

    o2 = o3.reshape(D_MODEL, SEQ)
    y = pl.pallas_call(
        _out_kernel,
        out_shape=jax.ShapeDtypeStruct((D_MODEL, SEQ), jnp.float32),
    )(Wm_p, o2, bm.reshape(D_MODEL, 1))
    return y[None]
